# baseline (speedup 1.0000x reference)
import numpy as np
import jax
import jax.numpy as jnp
from jax.experimental import pallas as pl
from jax.experimental.pallas import tpu as pltpu

_DEPTH = 14
_N = 2**_DEPTH - 1
_BATCH = 8
_F = 128
_PK = 2048
_NG = 2048


def _preorder_tables():
    perm = np.empty(_N, dtype=np.int64)
    stack = [0]
    k = 0
    while stack:
        i = stack.pop()
        perm[k] = i
        k += 1
        r, l = 2 * i + 2, 2 * i + 1
        if r < _N:
            stack.append(r)
        if l < _N:
            stack.append(l)
    pos = np.empty(_N, dtype=np.int64)
    pos[perm] = np.arange(_N)
    sub_base = pos[2047:4095]
    s = np.arange(_NG)
    assert np.all(pos[4095 + 2 * s] == sub_base + 1)
    assert np.all(pos[8191 + 4 * s] == sub_base + 2)
    assert np.all(pos[8192 + 4 * s] == sub_base + 3)
    assert np.all(pos[4096 + 2 * s] == sub_base + 4)
    assert np.all(pos[8193 + 4 * s] == sub_base + 5)
    assert np.all(pos[8194 + 4 * s] == sub_base + 6)
    assert np.all(sub_base[1::2] == sub_base[0::2] + 7)
    return [int(p) for p in pos[:2047]], [int(p) for p in sub_base[0::2]]


_UPPER_POS, _RUN14 = _preorder_tables()


def _tree_kernel(x_hbm, wu_ref, bu_ref, c2_ref, la_ref, lb_ref,
                 o_hbm, xbuf, ybuf, h_scr, y_scr, xsem, ysem):
    wu = wu_ref[...]
    bu = bu_ref[...]
    f32 = jnp.float32
    b = pl.program_id(0)
    slot = b % 2
    nslot = 1 - slot

    def xcopy(i, sl):
        return pltpu.make_async_copy(
            x_hbm.at[i], xbuf.at[sl].at[0:_N], xsem.at[sl])

    def ycopy(i, sl):
        return pltpu.make_async_copy(
            ybuf.at[sl].at[0:_N], o_hbm.at[i], ysem.at[sl])

    @pl.when(b == 0)
    def _():
        xcopy(0, 0).start()

    @pl.when(b < _BATCH - 1)
    def _():
        xcopy(b + 1, nslot).start()

    xcopy(b, slot).wait()

    def udot(src):
        return jnp.dot(src, wu, preferred_element_type=f32) + bu

    end = _PK + 7 * _NG
    xv = xbuf.at[slot]

    h_scr[0:2047, :] = udot(xv[0:2047, :])
    h_scr[_PK:end:7, :] = udot(xv[2047:4095, :])
    h_scr[_PK + 1:end:7, :] = udot(xv[4095:8191:2, :])
    h_scr[_PK + 4:end:7, :] = udot(xv[4096:8192:2, :])
    h_scr[_PK + 2:end:7, :] = udot(xv[8191:16383:4, :])
    h_scr[_PK + 3:end:7, :] = udot(xv[8192:16383:4, :])
    h_scr[_PK + 5:end:7, :] = udot(xv[8193:16383:4, :])
    h_scr[_PK + 6:end:7, :] = udot(xv[8194:16383:4, :])

    lamA = la_ref[...]
    lamB = lb_ref[...]

    def comb(children_sum):
        return lamA * children_sum + lamB * pltpu.roll(children_sum, 64, axis=1)

    def psl(o, st=7):
        return (slice(_PK + o, end, st), slice(None))

    h_scr[psl(1)] = h_scr[psl(1)] + comb(h_scr[psl(2)] + h_scr[psl(3)])
    h_scr[psl(4)] = h_scr[psl(4)] + comb(h_scr[psl(5)] + h_scr[psl(6)])
    h_scr[psl(0)] = h_scr[psl(0)] + comb(h_scr[psl(1)] + h_scr[psl(4)])
    h_scr[1023:2047, :] = (h_scr[1023:2047, :]
                           + comb(h_scr[psl(0, 14)] + h_scr[psl(7, 14)]))
    for d in range(9, -1, -1):
        m = 1 << d
        cs = 2 * m - 1
        s = h_scr[cs:cs + 2 * m:2, :] + h_scr[cs + 1:cs + 2 * m + 1:2, :]
        h_scr[m - 1:2 * m - 1, :] = h_scr[m - 1:2 * m - 1, :] + comb(s)

    y_scr[...] = jnp.dot(h_scr[...], c2_ref[...], preferred_element_type=f32)

    @pl.when(b >= 2)
    def _():
        ycopy(b - 2, slot).wait()

    ov = ybuf.at[slot]
    for n in range(2047):
        p = _UPPER_POS[n]
        ov[p:p + 1, :] = y_scr[n:n + 1, :]
    for t in range(1024):
        b14 = _RUN14[t]
        src = _PK + 14 * t
        ov[b14:b14 + 14, :] = y_scr[src:src + 14, :]

    ycopy(b, slot).start()

    @pl.when(b == _BATCH - 1)
    def _():
        ycopy(b - 1, nslot).wait()
        ycopy(b, slot).wait()


def kernel(x, Wp, bp, nu_log, theta_log, gamma_log, B_re, B_im, C_re, C_im):
    f32 = jnp.float32
    Lam_mod = jnp.exp(-jnp.exp(nu_log))
    theta = jnp.exp(theta_log)
    lre = Lam_mod * jnp.cos(theta)
    lim = Lam_mod * jnp.sin(theta)
    gamma = jnp.exp(gamma_log)
    hi = jax.lax.Precision.HIGHEST
    U2 = jnp.concatenate(
        [(gamma[:, None] * B_re).T, (gamma[:, None] * B_im).T], axis=1)
    W_u = jnp.dot(Wp.T, U2, precision=hi)
    b_u = jnp.dot(bp[None, :], U2, precision=hi)
    C2 = jnp.concatenate([C_re.T, -C_im.T], axis=0)
    lamA = jnp.concatenate([lre, lre])[None, :]
    lamB = jnp.concatenate([-lim, lim])[None, :]

    return pl.pallas_call(
        _tree_kernel,
        out_shape=jax.ShapeDtypeStruct((_BATCH, _N, _F), f32),
        grid=(_BATCH,),
        in_specs=[
            pl.BlockSpec(memory_space=pl.ANY),
            pl.BlockSpec((_F, _F), lambda b: (0, 0)),
            pl.BlockSpec((1, _F), lambda b: (0, 0)),
            pl.BlockSpec((_F, _F), lambda b: (0, 0)),
            pl.BlockSpec((1, _F), lambda b: (0, 0)),
            pl.BlockSpec((1, _F), lambda b: (0, 0)),
        ],
        out_specs=pl.BlockSpec(memory_space=pl.ANY),
        scratch_shapes=[
            pltpu.VMEM((2, _N + 1, _F), f32),
            pltpu.VMEM((2, _N + 1, _F), f32),
            pltpu.VMEM((_N + 1, _F), f32),
            pltpu.VMEM((_N + 1, _F), f32),
            pltpu.SemaphoreType.DMA((2,)),
            pltpu.SemaphoreType.DMA((2,)),
        ],
        compiler_params=pltpu.CompilerParams(
            dimension_semantics=("arbitrary",),
            vmem_limit_bytes=56 * 1024 * 1024),
        name="tree_lru",
    )(x, W_u, b_u, C2, lamA, lamB)


# speedup vs baseline: 1.0125x; 1.0125x over previous
import numpy as np
import jax
import jax.numpy as jnp
from jax.experimental import pallas as pl
from jax.experimental.pallas import tpu as pltpu

_DEPTH = 14
_N = 2**_DEPTH - 1
_BATCH = 8
_F = 128
_PK = 2048
_NG = 2048


def _preorder_tables():
    perm = np.empty(_N, dtype=np.int64)
    stack = [0]
    k = 0
    while stack:
        i = stack.pop()
        perm[k] = i
        k += 1
        r, l = 2 * i + 2, 2 * i + 1
        if r < _N:
            stack.append(r)
        if l < _N:
            stack.append(l)
    pos = np.empty(_N, dtype=np.int64)
    pos[perm] = np.arange(_N)
    sub_base = pos[2047:4095]
    s = np.arange(_NG)
    assert np.all(pos[4095 + 2 * s] == sub_base + 1)
    assert np.all(pos[8191 + 4 * s] == sub_base + 2)
    assert np.all(pos[8192 + 4 * s] == sub_base + 3)
    assert np.all(pos[4096 + 2 * s] == sub_base + 4)
    assert np.all(pos[8193 + 4 * s] == sub_base + 5)
    assert np.all(pos[8194 + 4 * s] == sub_base + 6)
    assert np.all(sub_base[1::2] == sub_base[0::2] + 7)
    return [int(p) for p in pos[:2047]], [int(p) for p in sub_base[0::2]]


_UPPER_POS, _RUN14 = _preorder_tables()


def _tree_kernel(x_ref, wu_ref, bu_ref, c2_ref, la_ref, lb_ref,
                 o_ref, h_scr, y_scr):
    wu = wu_ref[...]
    bu = bu_ref[...]
    f32 = jnp.float32

    def udot(src):
        return jnp.dot(src, wu, preferred_element_type=f32) + bu

    end = _PK + 7 * _NG
    xv = x_ref.at[0]

    h_scr[0:2047, :] = udot(xv[0:2047, :])
    h_scr[_PK:end:7, :] = udot(xv[2047:4095, :])
    h_scr[_PK + 1:end:7, :] = udot(xv[4095:8191:2, :])
    h_scr[_PK + 4:end:7, :] = udot(xv[4096:8192:2, :])
    h_scr[_PK + 2:end:7, :] = udot(xv[8191:16383:4, :])
    h_scr[_PK + 3:end:7, :] = udot(xv[8192:16383:4, :])
    h_scr[_PK + 5:end:7, :] = udot(xv[8193:16383:4, :])
    h_scr[_PK + 6:end:7, :] = udot(xv[8194:16383:4, :])

    lamA = la_ref[...]
    lamB = lb_ref[...]

    def comb(children_sum):
        return lamA * children_sum + lamB * pltpu.roll(children_sum, 64, axis=1)

    def psl(o, st=7):
        return (slice(_PK + o, end, st), slice(None))

    h_scr[psl(1)] = h_scr[psl(1)] + comb(h_scr[psl(2)] + h_scr[psl(3)])
    h_scr[psl(4)] = h_scr[psl(4)] + comb(h_scr[psl(5)] + h_scr[psl(6)])
    h_scr[psl(0)] = h_scr[psl(0)] + comb(h_scr[psl(1)] + h_scr[psl(4)])
    h_scr[1023:2047, :] = (h_scr[1023:2047, :]
                           + comb(h_scr[psl(0, 14)] + h_scr[psl(7, 14)]))
    for d in range(9, -1, -1):
        m = 1 << d
        cs = 2 * m - 1
        s = h_scr[cs:cs + 2 * m:2, :] + h_scr[cs + 1:cs + 2 * m + 1:2, :]
        h_scr[m - 1:2 * m - 1, :] = h_scr[m - 1:2 * m - 1, :] + comb(s)

    y_scr[...] = jnp.dot(h_scr[...], c2_ref[...], preferred_element_type=f32)

    ov = o_ref.at[0]
    for n in range(2047):
        p = _UPPER_POS[n]
        ov[p:p + 1, :] = y_scr[n:n + 1, :]
    for t in range(1024):
        b = _RUN14[t]
        src = _PK + 14 * t
        ov[b:b + 14, :] = y_scr[src:src + 14, :]


def kernel(x, Wp, bp, nu_log, theta_log, gamma_log, B_re, B_im, C_re, C_im):
    f32 = jnp.float32
    Lam_mod = jnp.exp(-jnp.exp(nu_log))
    theta = jnp.exp(theta_log)
    lre = Lam_mod * jnp.cos(theta)
    lim = Lam_mod * jnp.sin(theta)
    gamma = jnp.exp(gamma_log)
    hi = jax.lax.Precision.HIGHEST
    U2 = jnp.concatenate(
        [(gamma[:, None] * B_re).T, (gamma[:, None] * B_im).T], axis=1)
    W_u = jnp.dot(Wp.T, U2, precision=hi)
    b_u = jnp.dot(bp[None, :], U2, precision=hi)
    C2 = jnp.concatenate([C_re.T, -C_im.T], axis=0)
    lamA = jnp.concatenate([lre, lre])[None, :]
    lamB = jnp.concatenate([-lim, lim])[None, :]

    return pl.pallas_call(
        _tree_kernel,
        out_shape=jax.ShapeDtypeStruct((_BATCH, _N, _F), f32),
        grid=(_BATCH,),
        in_specs=[
            pl.BlockSpec((1, _N, _F), lambda b: (b, 0, 0)),
            pl.BlockSpec((_F, _F), lambda b: (0, 0)),
            pl.BlockSpec((1, _F), lambda b: (0, 0)),
            pl.BlockSpec((_F, _F), lambda b: (0, 0)),
            pl.BlockSpec((1, _F), lambda b: (0, 0)),
            pl.BlockSpec((1, _F), lambda b: (0, 0)),
        ],
        out_specs=pl.BlockSpec((1, _N, _F), lambda b: (b, 0, 0)),
        scratch_shapes=[
            pltpu.VMEM((_N + 1, _F), f32),
            pltpu.VMEM((_N + 1, _F), f32),
        ],
        compiler_params=pltpu.CompilerParams(
            dimension_semantics=("arbitrary",),
            vmem_limit_bytes=56 * 1024 * 1024),
        name="tree_lru",
    )(x, W_u, b_u, C2, lamA, lamB)
